# baseline (speedup 1.0000x reference)
import functools

import jax
import jax.numpy as jnp
from jax import lax
from jax.experimental import pallas as pl
from jax.experimental.pallas import tpu as pltpu
from jax.experimental.pallas import tpu_sc as plsc

_BSZ = 128
_MIX = 2
_FEAT = 128
_K = 2048
_TEMP = 0.07
_EPS = 0.1
_MOM = 0.5
_ITERS = 100

_NW = 32
_TOT = _BSZ * _K
_PER_W = _TOT // _NW
_CH = 128
_B_PER_W = _BSZ // _NW
_CH_PER_B = _K // _CH


def _mesh():
    return plsc.VectorSubcoreMesh(core_axis_name="c", subcore_axis_name="s")


def _sc_midx_body(mem_hbm, idx_hbm, midx_hbm, idxbuf, rows, sem):
    c = lax.axis_index("c")
    s = lax.axis_index("s")
    wid = s * 2 + c

    @pl.when(wid < 16)
    def _():
        base = pl.multiple_of(wid * 8, 8)
        pltpu.sync_copy(idx_hbm.at[pl.ds(base, 8)], idxbuf)
        pltpu.async_copy(mem_hbm.at[idxbuf], rows, sem).wait()
        pltpu.sync_copy(rows, midx_hbm.at[pl.ds(base, 8)])


def _sc_midx(memory, idx):
    f = pl.kernel(
        _sc_midx_body,
        mesh=_mesh(),
        out_type=jax.ShapeDtypeStruct((_BSZ, _FEAT), jnp.float32),
        scratch_types=[
            pltpu.VMEM((8,), jnp.int32),
            pltpu.VMEM((8, _FEAT), jnp.float32),
            pltpu.SemaphoreType.DMA,
        ],
    )
    return f(memory, idx)


def _lanes(val):
    return jnp.zeros((16,), jnp.int32) + val


def _sc_negdot_body(mem_hbm, cidx_hbm, es_hbm, negA_hbm, negB_hbm,
                    idxbuf, rows, es0, es1, outA, outB, sem):
    c = lax.axis_index("c")
    s = lax.axis_index("s")
    wid = s * 2 + c
    iota = lax.iota(jnp.int32, 16)
    zeros16 = jnp.zeros((16,), jnp.int32)

    def start(j):
        p = lax.rem(j, 2)
        isl = idxbuf.at[pl.ds(p * _CH, _CH)]
        pltpu.sync_copy(cidx_hbm.at[pl.ds(_chunk_off(j), _CH)], isl)
        pltpu.async_copy(mem_hbm.at[isl], rows.at[pl.ds(p * _CH, _CH)],
                         sem.at[p])

    def wait(j):
        p = lax.rem(j, 2)
        pltpu.make_async_copy(mem_hbm.at[pl.ds(0, _CH)],
                              rows.at[pl.ds(p * _CH, _CH)], sem.at[p]).wait()

    for t in range(_B_PER_W):
        b = wid * _B_PER_W + t
        base_chunk = b * _CH_PER_B

        def _chunk_off(j, _bc=base_chunk):
            return pl.multiple_of((_bc + j) * _CH, _CH)

        pltpu.sync_copy(es_hbm.at[pl.ds(2 * b * _FEAT, _FEAT)], es0)
        pltpu.sync_copy(es_hbm.at[pl.ds((2 * b + 1) * _FEAT, _FEAT)], es1)

        start(jnp.int32(0))

        def chunk_body(j, carry, _off=_chunk_off):
            @pl.when(j < _CH_PER_B - 1)
            def _():
                start(j + 1)
            wait(j)
            p = lax.rem(j, 2)
            rbase = p * _CH

            ridx = [rbase + g * 16 + iota for g in range(8)]

            def d_body(d, accs):
                a0 = accs[0]
                a1 = accs[1]
                dl = _lanes(d)
                e0 = plsc.load_gather(es0, [dl])
                e1 = plsc.load_gather(es1, [dl])
                na0 = []
                na1 = []
                for g in range(8):
                    col = plsc.load_gather(rows, [ridx[g], dl])
                    na0.append(a0[g] + col * e0)
                    na1.append(a1[g] + col * e1)
                return (tuple(na0), tuple(na1))

            z = tuple(jnp.zeros((16,), jnp.float32) for _ in range(8))
            acc0, acc1 = lax.fori_loop(0, _FEAT, d_body, (z, z))
            for g in range(8):
                outA[pl.ds(g * 16, 16)] = acc0[g]
                outB[pl.ds(g * 16, 16)] = acc1[g]
            pltpu.sync_copy(outA, negA_hbm.at[pl.ds(_off(j), _CH)])
            pltpu.sync_copy(outB, negB_hbm.at[pl.ds(_off(j), _CH)])
            return carry

        lax.fori_loop(0, _CH_PER_B, chunk_body, 0)


def _sc_negdot(memory, cidx_flat, es):
    f = pl.kernel(
        _sc_negdot_body,
        mesh=_mesh(),
        compiler_params=pltpu.CompilerParams(needs_layout_passes=False),
        out_type=[
            jax.ShapeDtypeStruct((_TOT,), jnp.float32),
            jax.ShapeDtypeStruct((_TOT,), jnp.float32),
        ],
        scratch_types=[
            pltpu.VMEM((2 * _CH,), jnp.int32),
            pltpu.VMEM((2 * _CH, _FEAT), jnp.float32),
            pltpu.VMEM((_FEAT,), jnp.float32),
            pltpu.VMEM((_FEAT,), jnp.float32),
            pltpu.VMEM((_CH,), jnp.float32),
            pltpu.VMEM((_CH,), jnp.float32),
            pltpu.SemaphoreType.DMA((2,)),
        ],
    )
    return f(memory, cidx_flat, es)


def _dotT(a, b):
    return lax.dot_general(a, b, (((1,), (1,)), ((), ())),
                           preferred_element_type=jnp.float32)


def _tca_body(fs_ref, ft_ref, wsw_ref, wsb_ref, wtw_ref, wtb_ref,
              midx_ref, idxr_ref, idxc_ref,
              es_ref, d_ref, posx_ref):
    fs = fs_ref[...]
    ft = ft_ref[...]
    es = _dotT(fs, wsw_ref[...]) + wsb_ref[...]
    et = _dotT(ft, wtw_ref[...]) + wtb_ref[...]
    es = es * jax.lax.rsqrt(jnp.sum(es * es, axis=1, keepdims=True))
    et = et * jax.lax.rsqrt(jnp.sum(et * et, axis=1, keepdims=True))

    ii = lax.broadcasted_iota(jnp.int32, (_BSZ, _BSZ * _MIX), 0)
    jj = lax.broadcasted_iota(jnp.int32, (_BSZ, _BSZ * _MIX), 1)
    sel_e = (jj == 2 * ii).astype(jnp.float32)
    sel_o = (jj == 2 * ii + 1).astype(jnp.float32)
    es_e = lax.dot_general(sel_e, es, (((1,), (0,)), ((), ())),
                           preferred_element_type=jnp.float32)
    es_o = lax.dot_general(sel_o, es, (((1,), (0,)), ((), ())),
                           preferred_element_type=jnp.float32)
    et_e = lax.dot_general(sel_e, et, (((1,), (0,)), ((), ())),
                           preferred_element_type=jnp.float32)
    et_o = lax.dot_general(sel_o, et, (((1,), (0,)), ((), ())),
                           preferred_element_type=jnp.float32)

    midx = midx_ref[...]
    pos = midx * _MOM + et_e * (1.0 - _MOM)
    pos = pos * jax.lax.rsqrt(jnp.sum(pos * pos, axis=1, keepdims=True))

    idx_r = idxr_ref[...]
    idx_c = idxc_ref[...]
    eqm = (idx_c == idx_r).astype(jnp.float32)
    pp = lax.broadcasted_iota(jnp.int32, (_BSZ, _BSZ), 0)
    qq = lax.broadcasted_iota(jnp.int32, (_BSZ, _BSZ), 1)
    later_dup = eqm * (qq > pp).astype(jnp.float32)
    active = 1.0 - jnp.max(later_dup, axis=1, keepdims=True)

    delta = (pos - midx) * active
    d_ref[...] = _dotT(es, delta)

    g00 = jnp.sum(es_e * et_e, axis=1, keepdims=True)
    g01 = jnp.sum(es_e * et_o, axis=1, keepdims=True)
    g10 = jnp.sum(es_o * et_e, axis=1, keepdims=True)
    g11 = jnp.sum(es_o * et_o, axis=1, keepdims=True)
    c00 = 2.0 - 2.0 * g00
    c01 = 2.0 - 2.0 * g01
    c10 = 2.0 - 2.0 * g10
    c11 = 2.0 - 2.0 * g11
    lmu = jnp.log(0.5 + 1e-8)

    def m_all(u0, u1, v0, v1):
        m00 = (-c00 + u0 + v0) / _EPS
        m01 = (-c01 + u0 + v1) / _EPS
        m10 = (-c10 + u1 + v0) / _EPS
        m11 = (-c11 + u1 + v1) / _EPS
        return m00, m01, m10, m11

    def sink_step(_, carry):
        u0, u1, v0, v1 = carry
        m00, m01, m10, m11 = m_all(u0, u1, v0, v1)
        u0 = _EPS * (lmu - jnp.logaddexp(m00, m01)) + u0
        u1 = _EPS * (lmu - jnp.logaddexp(m10, m11)) + u1
        m00, m01, m10, m11 = m_all(u0, u1, v0, v1)
        v0 = _EPS * (lmu - jnp.logaddexp(m00, m10)) + v0
        v1 = _EPS * (lmu - jnp.logaddexp(m01, m11)) + v1
        return u0, u1, v0, v1

    z = jnp.zeros((_BSZ, 1), jnp.float32)
    u0, u1, v0, v1 = lax.fori_loop(0, _ITERS, sink_step, (z, z, z, z))
    m00, m01, m10, m11 = m_all(u0, u1, v0, v1)
    posx = (jnp.exp(m00) * g00 + jnp.exp(m01) * g01 +
            jnp.exp(m10) * g10 + jnp.exp(m11) * g11)

    es_ref[...] = es
    posx_ref[...] = posx


def _tca(feat_s, feat_t, wsw, wsb, wtw, wtb, midx, idx_row, idx_col):
    return pl.pallas_call(
        _tca_body,
        out_shape=[
            jax.ShapeDtypeStruct((_BSZ * _MIX, _FEAT), jnp.float32),
            jax.ShapeDtypeStruct((_BSZ * _MIX, _BSZ), jnp.float32),
            jax.ShapeDtypeStruct((_BSZ, 1), jnp.float32),
        ],
    )(feat_s, feat_t, wsw, wsb, wtw, wtb, midx, idx_row, idx_col)


def _tcb_body(negA_ref, negB_ref, d2_ref, cidx_ref, idxc_ref, ng_ref):
    b = pl.program_id(0)
    negA = negA_ref[0]
    negB = negB_ref[0]
    d_pair = d2_ref[0]
    cidx_row = cidx_ref[0]
    idx_col = idxc_ref[...]

    eq = (idx_col == cidx_row).astype(jnp.float32)
    corr = lax.dot_general(d_pair, eq, (((1,), (0,)), ((), ())),
                           preferred_element_type=jnp.float32)
    ng = (jnp.exp((negA + corr[0:1, :]) / _TEMP) +
          jnp.exp((negB + corr[1:2, :]) / _TEMP))
    ng_ref[0, b] = jnp.sum(ng)


def _tcb(negA3, negB3, d2, cidx3, idx_col):
    return pl.pallas_call(
        _tcb_body,
        grid=(_BSZ,),
        in_specs=[
            pl.BlockSpec((1, 1, _K), lambda b: (b, 0, 0)),
            pl.BlockSpec((1, 1, _K), lambda b: (b, 0, 0)),
            pl.BlockSpec((1, _MIX, _BSZ), lambda b: (b, 0, 0)),
            pl.BlockSpec((1, 1, _K), lambda b: (b, 0, 0)),
            pl.BlockSpec((_BSZ, 1), lambda b: (0, 0)),
        ],
        out_specs=pl.BlockSpec(memory_space=pltpu.SMEM),
        out_shape=jax.ShapeDtypeStruct((1, _BSZ), jnp.float32),
    )(negA3, negB3, d2, cidx3, idx_col)


def _tcc_body(posx_ref, ng_ref, out_ref):
    p = jnp.exp(posx_ref[...] / _TEMP)
    ngs = ng_ref[...]
    logits = jnp.log(p / (p + ngs))
    out_ref[0, 0] = -jnp.sum(logits) / _BSZ


def _tcc(posx_row, ng_row):
    return pl.pallas_call(
        _tcc_body,
        out_specs=pl.BlockSpec(memory_space=pltpu.SMEM),
        out_shape=jax.ShapeDtypeStruct((1, 1), jnp.float32),
    )(posx_row, ng_row)


def kernel(feat_s, feat_t, memory, Ws_w, Ws_b, Wt_w, Wt_b, labels, idx,
           contrast_idx):
    feat_s = feat_s.reshape(_BSZ * _MIX, -1)
    feat_t = feat_t.reshape(_BSZ * _MIX, -1)
    cidx_flat = contrast_idx.reshape(_TOT).astype(jnp.int32)
    idx_i = idx.astype(jnp.int32)

    midx = _sc_midx(memory, idx_i)

    idx_f = idx.astype(jnp.float32)
    idx_row = idx_f.reshape(1, _BSZ)
    idx_col = idx_f.reshape(_BSZ, 1)
    es, dmat, posx = _tca(feat_s, feat_t, Ws_w, Ws_b.reshape(1, _FEAT),
                          Wt_w, Wt_b.reshape(1, _FEAT), midx, idx_row, idx_col)

    negA, negB = _sc_negdot(memory, cidx_flat, es.reshape(_BSZ * _MIX * _FEAT))

    negA3 = negA.reshape(_BSZ, 1, _K)
    negB3 = negB.reshape(_BSZ, 1, _K)
    d2 = dmat.reshape(_BSZ, _MIX, _BSZ)
    cidx3 = contrast_idx.astype(jnp.float32).reshape(_BSZ, 1, _K)
    ng_row = _tcb(negA3, negB3, d2, cidx3, idx_col)

    loss = _tcc(posx.reshape(1, _BSZ), ng_row)
    return loss.reshape(())


# speedup vs baseline: 1.0044x; 1.0044x over previous
import functools

import jax
import jax.numpy as jnp
from jax import lax
from jax.experimental import pallas as pl
from jax.experimental.pallas import tpu as pltpu
from jax.experimental.pallas import tpu_sc as plsc

_BSZ = 128
_MIX = 2
_FEAT = 128
_K = 2048
_TEMP = 0.07
_EPS = 0.1
_MOM = 0.5
_ITERS = 100

_NW = 32
_TOT = _BSZ * _K
_PER_W = _TOT // _NW
_CH = 128
_B_PER_W = _BSZ // _NW
_CH_PER_B = _K // _CH


def _mesh():
    return plsc.VectorSubcoreMesh(core_axis_name="c", subcore_axis_name="s")


def _sc_midx_body(mem_hbm, idx_hbm, midx_hbm, idxbuf, rows, sem):
    c = lax.axis_index("c")
    s = lax.axis_index("s")
    wid = s * 2 + c

    @pl.when(wid < 16)
    def _():
        base = pl.multiple_of(wid * 8, 8)
        pltpu.sync_copy(idx_hbm.at[pl.ds(base, 8)], idxbuf)
        pltpu.async_copy(mem_hbm.at[idxbuf], rows, sem).wait()
        pltpu.sync_copy(rows, midx_hbm.at[pl.ds(base, 8)])


def _sc_midx(memory, idx):
    f = pl.kernel(
        _sc_midx_body,
        mesh=_mesh(),
        out_type=jax.ShapeDtypeStruct((_BSZ, _FEAT), jnp.float32),
        scratch_types=[
            pltpu.VMEM((8,), jnp.int32),
            pltpu.VMEM((8, _FEAT), jnp.float32),
            pltpu.SemaphoreType.DMA,
        ],
    )
    return f(memory, idx)


def _lanes(val):
    return jnp.zeros((16,), jnp.int32) + val


def _sc_negdot_body(mem_hbm, cidx_hbm, es_hbm, negA_hbm, negB_hbm,
                    idxbuf, rows, es0, es1, outA, outB, sem):
    c = lax.axis_index("c")
    s = lax.axis_index("s")
    wid = s * 2 + c
    iota = lax.iota(jnp.int32, 16)
    zeros16 = jnp.zeros((16,), jnp.int32)

    def start(j):
        p = lax.rem(j, 2)
        isl = idxbuf.at[pl.ds(p * _CH, _CH)]
        pltpu.sync_copy(cidx_hbm.at[pl.ds(_chunk_off(j), _CH)], isl)
        pltpu.async_copy(mem_hbm.at[isl], rows.at[pl.ds(p * _CH, _CH)],
                         sem.at[p])

    def wait(j):
        p = lax.rem(j, 2)
        pltpu.make_async_copy(mem_hbm.at[pl.ds(0, _CH)],
                              rows.at[pl.ds(p * _CH, _CH)], sem.at[p]).wait()

    for t in range(_B_PER_W):
        b = wid * _B_PER_W + t
        base_chunk = b * _CH_PER_B

        def _chunk_off(j, _bc=base_chunk):
            return pl.multiple_of((_bc + j) * _CH, _CH)

        pltpu.sync_copy(es_hbm.at[pl.ds(2 * b * _FEAT, _FEAT)], es0)
        pltpu.sync_copy(es_hbm.at[pl.ds((2 * b + 1) * _FEAT, _FEAT)], es1)

        start(jnp.int32(0))

        def chunk_body(j, carry, _off=_chunk_off):
            @pl.when(j < _CH_PER_B - 1)
            def _():
                start(j + 1)
            wait(j)
            p = lax.rem(j, 2)
            rbase = p * _CH

            ridx = [rbase + g * 16 + iota for g in range(8)]

            def d_body(d, accs):
                a0 = accs[0]
                a1 = accs[1]
                dl = _lanes(d)
                e0 = plsc.load_gather(es0, [dl])
                e1 = plsc.load_gather(es1, [dl])
                na0 = []
                na1 = []
                for g in range(8):
                    col = plsc.load_gather(rows, [ridx[g], dl])
                    na0.append(a0[g] + col * e0)
                    na1.append(a1[g] + col * e1)
                return (tuple(na0), tuple(na1))

            z = tuple(jnp.zeros((16,), jnp.float32) for _ in range(8))
            acc0, acc1 = plsc.parallel_loop(0, _FEAT, unroll=8,
                                            carry=(z, z))(d_body)
            for g in range(8):
                outA[pl.ds(g * 16, 16)] = acc0[g]
                outB[pl.ds(g * 16, 16)] = acc1[g]
            pltpu.sync_copy(outA, negA_hbm.at[pl.ds(_off(j), _CH)])
            pltpu.sync_copy(outB, negB_hbm.at[pl.ds(_off(j), _CH)])
            return carry

        lax.fori_loop(0, _CH_PER_B, chunk_body, 0)


def _sc_negdot(memory, cidx_flat, es):
    f = pl.kernel(
        _sc_negdot_body,
        mesh=_mesh(),
        compiler_params=pltpu.CompilerParams(needs_layout_passes=False),
        out_type=[
            jax.ShapeDtypeStruct((_TOT,), jnp.float32),
            jax.ShapeDtypeStruct((_TOT,), jnp.float32),
        ],
        scratch_types=[
            pltpu.VMEM((2 * _CH,), jnp.int32),
            pltpu.VMEM((2 * _CH, _FEAT), jnp.float32),
            pltpu.VMEM((_FEAT,), jnp.float32),
            pltpu.VMEM((_FEAT,), jnp.float32),
            pltpu.VMEM((_CH,), jnp.float32),
            pltpu.VMEM((_CH,), jnp.float32),
            pltpu.SemaphoreType.DMA((2,)),
        ],
    )
    return f(memory, cidx_flat, es)


def _dotT(a, b):
    return lax.dot_general(a, b, (((1,), (1,)), ((), ())),
                           preferred_element_type=jnp.float32)


def _tca_body(fs_ref, ft_ref, wsw_ref, wsb_ref, wtw_ref, wtb_ref,
              midx_ref, idxr_ref, idxc_ref,
              es_ref, d_ref, posx_ref):
    fs = fs_ref[...]
    ft = ft_ref[...]
    es = _dotT(fs, wsw_ref[...]) + wsb_ref[...]
    et = _dotT(ft, wtw_ref[...]) + wtb_ref[...]
    es = es * jax.lax.rsqrt(jnp.sum(es * es, axis=1, keepdims=True))
    et = et * jax.lax.rsqrt(jnp.sum(et * et, axis=1, keepdims=True))

    ii = lax.broadcasted_iota(jnp.int32, (_BSZ, _BSZ * _MIX), 0)
    jj = lax.broadcasted_iota(jnp.int32, (_BSZ, _BSZ * _MIX), 1)
    sel_e = (jj == 2 * ii).astype(jnp.float32)
    sel_o = (jj == 2 * ii + 1).astype(jnp.float32)
    es_e = lax.dot_general(sel_e, es, (((1,), (0,)), ((), ())),
                           preferred_element_type=jnp.float32)
    es_o = lax.dot_general(sel_o, es, (((1,), (0,)), ((), ())),
                           preferred_element_type=jnp.float32)
    et_e = lax.dot_general(sel_e, et, (((1,), (0,)), ((), ())),
                           preferred_element_type=jnp.float32)
    et_o = lax.dot_general(sel_o, et, (((1,), (0,)), ((), ())),
                           preferred_element_type=jnp.float32)

    midx = midx_ref[...]
    pos = midx * _MOM + et_e * (1.0 - _MOM)
    pos = pos * jax.lax.rsqrt(jnp.sum(pos * pos, axis=1, keepdims=True))

    idx_r = idxr_ref[...]
    idx_c = idxc_ref[...]
    eqm = (idx_c == idx_r).astype(jnp.float32)
    pp = lax.broadcasted_iota(jnp.int32, (_BSZ, _BSZ), 0)
    qq = lax.broadcasted_iota(jnp.int32, (_BSZ, _BSZ), 1)
    later_dup = eqm * (qq > pp).astype(jnp.float32)
    active = 1.0 - jnp.max(later_dup, axis=1, keepdims=True)

    delta = (pos - midx) * active
    d_ref[...] = _dotT(es, delta)

    g00 = jnp.sum(es_e * et_e, axis=1, keepdims=True)
    g01 = jnp.sum(es_e * et_o, axis=1, keepdims=True)
    g10 = jnp.sum(es_o * et_e, axis=1, keepdims=True)
    g11 = jnp.sum(es_o * et_o, axis=1, keepdims=True)
    c00 = 2.0 - 2.0 * g00
    c01 = 2.0 - 2.0 * g01
    c10 = 2.0 - 2.0 * g10
    c11 = 2.0 - 2.0 * g11
    lmu = jnp.log(0.5 + 1e-8)

    def m_all(u0, u1, v0, v1):
        m00 = (-c00 + u0 + v0) / _EPS
        m01 = (-c01 + u0 + v1) / _EPS
        m10 = (-c10 + u1 + v0) / _EPS
        m11 = (-c11 + u1 + v1) / _EPS
        return m00, m01, m10, m11

    def sink_step(_, carry):
        u0, u1, v0, v1 = carry
        m00, m01, m10, m11 = m_all(u0, u1, v0, v1)
        u0 = _EPS * (lmu - jnp.logaddexp(m00, m01)) + u0
        u1 = _EPS * (lmu - jnp.logaddexp(m10, m11)) + u1
        m00, m01, m10, m11 = m_all(u0, u1, v0, v1)
        v0 = _EPS * (lmu - jnp.logaddexp(m00, m10)) + v0
        v1 = _EPS * (lmu - jnp.logaddexp(m01, m11)) + v1
        return u0, u1, v0, v1

    z = jnp.zeros((_BSZ, 1), jnp.float32)
    u0, u1, v0, v1 = lax.fori_loop(0, _ITERS, sink_step, (z, z, z, z))
    m00, m01, m10, m11 = m_all(u0, u1, v0, v1)
    posx = (jnp.exp(m00) * g00 + jnp.exp(m01) * g01 +
            jnp.exp(m10) * g10 + jnp.exp(m11) * g11)

    es_ref[...] = es
    posx_ref[...] = posx


def _tca(feat_s, feat_t, wsw, wsb, wtw, wtb, midx, idx_row, idx_col):
    return pl.pallas_call(
        _tca_body,
        out_shape=[
            jax.ShapeDtypeStruct((_BSZ * _MIX, _FEAT), jnp.float32),
            jax.ShapeDtypeStruct((_BSZ * _MIX, _BSZ), jnp.float32),
            jax.ShapeDtypeStruct((_BSZ, 1), jnp.float32),
        ],
    )(feat_s, feat_t, wsw, wsb, wtw, wtb, midx, idx_row, idx_col)


def _tcb_body(negA_ref, negB_ref, d2_ref, cidx_ref, idxc_ref, ng_ref):
    b = pl.program_id(0)
    negA = negA_ref[0]
    negB = negB_ref[0]
    d_pair = d2_ref[0]
    cidx_row = cidx_ref[0]
    idx_col = idxc_ref[...]

    eq = (idx_col == cidx_row).astype(jnp.float32)
    corr = lax.dot_general(d_pair, eq, (((1,), (0,)), ((), ())),
                           preferred_element_type=jnp.float32)
    ng = (jnp.exp((negA + corr[0:1, :]) / _TEMP) +
          jnp.exp((negB + corr[1:2, :]) / _TEMP))
    ng_ref[0, b] = jnp.sum(ng)


def _tcb(negA3, negB3, d2, cidx3, idx_col):
    return pl.pallas_call(
        _tcb_body,
        grid=(_BSZ,),
        in_specs=[
            pl.BlockSpec((1, 1, _K), lambda b: (b, 0, 0)),
            pl.BlockSpec((1, 1, _K), lambda b: (b, 0, 0)),
            pl.BlockSpec((1, _MIX, _BSZ), lambda b: (b, 0, 0)),
            pl.BlockSpec((1, 1, _K), lambda b: (b, 0, 0)),
            pl.BlockSpec((_BSZ, 1), lambda b: (0, 0)),
        ],
        out_specs=pl.BlockSpec(memory_space=pltpu.SMEM),
        out_shape=jax.ShapeDtypeStruct((1, _BSZ), jnp.float32),
    )(negA3, negB3, d2, cidx3, idx_col)


def _tcc_body(posx_ref, ng_ref, out_ref):
    p = jnp.exp(posx_ref[...] / _TEMP)
    ngs = ng_ref[...]
    logits = jnp.log(p / (p + ngs))
    out_ref[0, 0] = -jnp.sum(logits) / _BSZ


def _tcc(posx_row, ng_row):
    return pl.pallas_call(
        _tcc_body,
        out_specs=pl.BlockSpec(memory_space=pltpu.SMEM),
        out_shape=jax.ShapeDtypeStruct((1, 1), jnp.float32),
    )(posx_row, ng_row)


def kernel(feat_s, feat_t, memory, Ws_w, Ws_b, Wt_w, Wt_b, labels, idx,
           contrast_idx):
    feat_s = feat_s.reshape(_BSZ * _MIX, -1)
    feat_t = feat_t.reshape(_BSZ * _MIX, -1)
    cidx_flat = contrast_idx.reshape(_TOT).astype(jnp.int32)
    idx_i = idx.astype(jnp.int32)

    midx = _sc_midx(memory, idx_i)

    idx_f = idx.astype(jnp.float32)
    idx_row = idx_f.reshape(1, _BSZ)
    idx_col = idx_f.reshape(_BSZ, 1)
    es, dmat, posx = _tca(feat_s, feat_t, Ws_w, Ws_b.reshape(1, _FEAT),
                          Wt_w, Wt_b.reshape(1, _FEAT), midx, idx_row, idx_col)

    negA, negB = _sc_negdot(memory, cidx_flat, es.reshape(_BSZ * _MIX * _FEAT))

    negA3 = negA.reshape(_BSZ, 1, _K)
    negB3 = negB.reshape(_BSZ, 1, _K)
    d2 = dmat.reshape(_BSZ, _MIX, _BSZ)
    cidx3 = contrast_idx.astype(jnp.float32).reshape(_BSZ, 1, _K)
    ng_row = _tcb(negA3, negB3, d2, cidx3, idx_col)

    loss = _tcc(posx.reshape(1, _BSZ), ng_row)
    return loss.reshape(())


# speedup vs baseline: 1.7530x; 1.7453x over previous
import functools

import jax
import jax.numpy as jnp
from jax import lax
from jax.experimental import pallas as pl
from jax.experimental.pallas import tpu as pltpu
from jax.experimental.pallas import tpu_sc as plsc

_BSZ = 128
_MIX = 2
_FEAT = 128
_K = 2048
_TEMP = 0.07
_EPS = 0.1
_MOM = 0.5
_ITERS = 100

_NW = 32
_TOT = _BSZ * _K
_PER_W = _TOT // _NW
_CH = 128
_B_PER_W = _BSZ // _NW
_CH_PER_B = _K // _CH


def _mesh():
    return plsc.VectorSubcoreMesh(core_axis_name="c", subcore_axis_name="s")


def _sc_midx_body(mem_hbm, idx_hbm, midx_hbm, idxbuf, rows, sem):
    c = lax.axis_index("c")
    s = lax.axis_index("s")
    wid = s * 2 + c

    @pl.when(wid < 16)
    def _():
        base = pl.multiple_of(wid * 8, 8)
        pltpu.sync_copy(idx_hbm.at[pl.ds(base, 8)], idxbuf)
        pltpu.async_copy(mem_hbm.at[idxbuf], rows, sem).wait()
        pltpu.sync_copy(rows, midx_hbm.at[pl.ds(base, 8)])


def _sc_midx(memory, idx):
    f = pl.kernel(
        _sc_midx_body,
        mesh=_mesh(),
        out_type=jax.ShapeDtypeStruct((_BSZ, _FEAT), jnp.float32),
        scratch_types=[
            pltpu.VMEM((8,), jnp.int32),
            pltpu.VMEM((8, _FEAT), jnp.float32),
            pltpu.SemaphoreType.DMA,
        ],
    )
    return f(memory, idx)


_NBLK = 2048
_NSTEPS = (100000 + _NBLK - 1) // _NBLK


def _tcd_body(es_ref, mem_ref, out_ref):
    out_ref[...] = _dotT(es_ref[...], mem_ref[...])


def _tcd(es, memory):
    return pl.pallas_call(
        _tcd_body,
        grid=(_NSTEPS,),
        in_specs=[
            pl.BlockSpec((_BSZ * _MIX, _FEAT), lambda i: (0, 0)),
            pl.BlockSpec((_NBLK, _FEAT), lambda i: (i, 0)),
        ],
        out_specs=pl.BlockSpec((_BSZ * _MIX, _NBLK), lambda i: (0, i)),
        out_shape=jax.ShapeDtypeStruct((_BSZ * _MIX, 100000), jnp.float32),
        compiler_params=pltpu.CompilerParams(
            dimension_semantics=("arbitrary",)),
    )(es, memory)


def _lanes(val):
    return jnp.zeros((16,), jnp.int32) + val


def _sc_neggather_body(lg_hbm, cidx_hbm, negA_hbm, negB_hbm,
                       cbuf, abuf, bbuf, rA, rB, semA, semB):
    c = lax.axis_index("c")
    s = lax.axis_index("s")
    wid = s * 2 + c

    for t in range(_B_PER_W):
        b = wid * _B_PER_W + t
        base_a = 2 * b * 100000
        base_chunk = b * _CH_PER_B

        def chunk_body(j, carry, _bc=base_chunk, _ba=base_a):
            off = pl.multiple_of((_bc + j) * _CH, _CH)
            pltpu.sync_copy(cidx_hbm.at[pl.ds(off, _CH)], cbuf)
            for g in range(8):
                v = cbuf[pl.ds(g * 16, 16)]
                abuf[pl.ds(g * 16, 16)] = v + _ba
                bbuf[pl.ds(g * 16, 16)] = v + (_ba + 100000)
            cpA = pltpu.async_copy(lg_hbm.at[abuf], rA, semA)
            cpB = pltpu.async_copy(lg_hbm.at[bbuf], rB, semB)
            cpA.wait()
            cpB.wait()
            pltpu.sync_copy(rA, negA_hbm.at[pl.ds(off, _CH)])
            pltpu.sync_copy(rB, negB_hbm.at[pl.ds(off, _CH)])
            return carry

        lax.fori_loop(0, _CH_PER_B, chunk_body, 0)


def _sc_neggather(logits_flat, cidx_flat):
    f = pl.kernel(
        _sc_neggather_body,
        mesh=_mesh(),
        out_type=[
            jax.ShapeDtypeStruct((_TOT,), jnp.float32),
            jax.ShapeDtypeStruct((_TOT,), jnp.float32),
        ],
        scratch_types=[
            pltpu.VMEM((_CH,), jnp.int32),
            pltpu.VMEM((_CH,), jnp.int32),
            pltpu.VMEM((_CH,), jnp.int32),
            pltpu.VMEM((_CH,), jnp.float32),
            pltpu.VMEM((_CH,), jnp.float32),
            pltpu.SemaphoreType.DMA,
            pltpu.SemaphoreType.DMA,
        ],
    )
    return f(logits_flat, cidx_flat)


def _dotT(a, b):
    return lax.dot_general(a, b, (((1,), (1,)), ((), ())),
                           preferred_element_type=jnp.float32)


def _tca_body(fs_ref, ft_ref, wsw_ref, wsb_ref, wtw_ref, wtb_ref,
              midx_ref, idxr_ref, idxc_ref,
              es_ref, d_ref, posx_ref):
    fs = fs_ref[...]
    ft = ft_ref[...]
    es = _dotT(fs, wsw_ref[...]) + wsb_ref[...]
    et = _dotT(ft, wtw_ref[...]) + wtb_ref[...]
    es = es * jax.lax.rsqrt(jnp.sum(es * es, axis=1, keepdims=True))
    et = et * jax.lax.rsqrt(jnp.sum(et * et, axis=1, keepdims=True))

    ii = lax.broadcasted_iota(jnp.int32, (_BSZ, _BSZ * _MIX), 0)
    jj = lax.broadcasted_iota(jnp.int32, (_BSZ, _BSZ * _MIX), 1)
    sel_e = (jj == 2 * ii).astype(jnp.float32)
    sel_o = (jj == 2 * ii + 1).astype(jnp.float32)
    es_e = lax.dot_general(sel_e, es, (((1,), (0,)), ((), ())),
                           preferred_element_type=jnp.float32)
    es_o = lax.dot_general(sel_o, es, (((1,), (0,)), ((), ())),
                           preferred_element_type=jnp.float32)
    et_e = lax.dot_general(sel_e, et, (((1,), (0,)), ((), ())),
                           preferred_element_type=jnp.float32)
    et_o = lax.dot_general(sel_o, et, (((1,), (0,)), ((), ())),
                           preferred_element_type=jnp.float32)

    midx = midx_ref[...]
    pos = midx * _MOM + et_e * (1.0 - _MOM)
    pos = pos * jax.lax.rsqrt(jnp.sum(pos * pos, axis=1, keepdims=True))

    idx_r = idxr_ref[...]
    idx_c = idxc_ref[...]
    eqm = (idx_c == idx_r).astype(jnp.float32)
    pp = lax.broadcasted_iota(jnp.int32, (_BSZ, _BSZ), 0)
    qq = lax.broadcasted_iota(jnp.int32, (_BSZ, _BSZ), 1)
    later_dup = eqm * (qq > pp).astype(jnp.float32)
    active = 1.0 - jnp.max(later_dup, axis=1, keepdims=True)

    delta = (pos - midx) * active
    d_ref[...] = _dotT(es, delta)

    g00 = jnp.sum(es_e * et_e, axis=1, keepdims=True)
    g01 = jnp.sum(es_e * et_o, axis=1, keepdims=True)
    g10 = jnp.sum(es_o * et_e, axis=1, keepdims=True)
    g11 = jnp.sum(es_o * et_o, axis=1, keepdims=True)
    c00 = 2.0 - 2.0 * g00
    c01 = 2.0 - 2.0 * g01
    c10 = 2.0 - 2.0 * g10
    c11 = 2.0 - 2.0 * g11
    lmu = jnp.log(0.5 + 1e-8)

    def m_all(u0, u1, v0, v1):
        m00 = (-c00 + u0 + v0) / _EPS
        m01 = (-c01 + u0 + v1) / _EPS
        m10 = (-c10 + u1 + v0) / _EPS
        m11 = (-c11 + u1 + v1) / _EPS
        return m00, m01, m10, m11

    def sink_step(_, carry):
        u0, u1, v0, v1 = carry
        m00, m01, m10, m11 = m_all(u0, u1, v0, v1)
        u0 = _EPS * (lmu - jnp.logaddexp(m00, m01)) + u0
        u1 = _EPS * (lmu - jnp.logaddexp(m10, m11)) + u1
        m00, m01, m10, m11 = m_all(u0, u1, v0, v1)
        v0 = _EPS * (lmu - jnp.logaddexp(m00, m10)) + v0
        v1 = _EPS * (lmu - jnp.logaddexp(m01, m11)) + v1
        return u0, u1, v0, v1

    z = jnp.zeros((_BSZ, 1), jnp.float32)
    u0, u1, v0, v1 = lax.fori_loop(0, _ITERS, sink_step, (z, z, z, z))
    m00, m01, m10, m11 = m_all(u0, u1, v0, v1)
    posx = (jnp.exp(m00) * g00 + jnp.exp(m01) * g01 +
            jnp.exp(m10) * g10 + jnp.exp(m11) * g11)

    es_ref[...] = es
    posx_ref[...] = posx


def _tca(feat_s, feat_t, wsw, wsb, wtw, wtb, midx, idx_row, idx_col):
    return pl.pallas_call(
        _tca_body,
        out_shape=[
            jax.ShapeDtypeStruct((_BSZ * _MIX, _FEAT), jnp.float32),
            jax.ShapeDtypeStruct((_BSZ * _MIX, _BSZ), jnp.float32),
            jax.ShapeDtypeStruct((_BSZ, 1), jnp.float32),
        ],
    )(feat_s, feat_t, wsw, wsb, wtw, wtb, midx, idx_row, idx_col)


def _tcb_body(negA_ref, negB_ref, d2_ref, cidx_ref, idxc_ref, ng_ref):
    b = pl.program_id(0)
    negA = negA_ref[0]
    negB = negB_ref[0]
    d_pair = d2_ref[0]
    cidx_row = cidx_ref[0]
    idx_col = idxc_ref[...]

    eq = (idx_col == cidx_row).astype(jnp.float32)
    corr = lax.dot_general(d_pair, eq, (((1,), (0,)), ((), ())),
                           preferred_element_type=jnp.float32)
    ng = (jnp.exp((negA + corr[0:1, :]) / _TEMP) +
          jnp.exp((negB + corr[1:2, :]) / _TEMP))
    ng_ref[0, b] = jnp.sum(ng)


def _tcb(negA3, negB3, d2, cidx3, idx_col):
    return pl.pallas_call(
        _tcb_body,
        grid=(_BSZ,),
        in_specs=[
            pl.BlockSpec((1, 1, _K), lambda b: (b, 0, 0)),
            pl.BlockSpec((1, 1, _K), lambda b: (b, 0, 0)),
            pl.BlockSpec((1, _MIX, _BSZ), lambda b: (b, 0, 0)),
            pl.BlockSpec((1, 1, _K), lambda b: (b, 0, 0)),
            pl.BlockSpec((_BSZ, 1), lambda b: (0, 0)),
        ],
        out_specs=pl.BlockSpec(memory_space=pltpu.SMEM),
        out_shape=jax.ShapeDtypeStruct((1, _BSZ), jnp.float32),
    )(negA3, negB3, d2, cidx3, idx_col)


def _tcc_body(posx_ref, ng_ref, out_ref):
    p = jnp.exp(posx_ref[...] / _TEMP)
    ngs = ng_ref[...]
    logits = jnp.log(p / (p + ngs))
    out_ref[0, 0] = -jnp.sum(logits) / _BSZ


def _tcc(posx_row, ng_row):
    return pl.pallas_call(
        _tcc_body,
        out_specs=pl.BlockSpec(memory_space=pltpu.SMEM),
        out_shape=jax.ShapeDtypeStruct((1, 1), jnp.float32),
    )(posx_row, ng_row)


def kernel(feat_s, feat_t, memory, Ws_w, Ws_b, Wt_w, Wt_b, labels, idx,
           contrast_idx):
    feat_s = feat_s.reshape(_BSZ * _MIX, -1)
    feat_t = feat_t.reshape(_BSZ * _MIX, -1)
    cidx_flat = contrast_idx.reshape(_TOT).astype(jnp.int32)
    idx_i = idx.astype(jnp.int32)

    midx = _sc_midx(memory, idx_i)

    idx_f = idx.astype(jnp.float32)
    idx_row = idx_f.reshape(1, _BSZ)
    idx_col = idx_f.reshape(_BSZ, 1)
    es, dmat, posx = _tca(feat_s, feat_t, Ws_w, Ws_b.reshape(1, _FEAT),
                          Wt_w, Wt_b.reshape(1, _FEAT), midx, idx_row, idx_col)

    logits = _tcd(es, memory)
    negA, negB = _sc_neggather(logits.reshape(_BSZ * _MIX * 100000), cidx_flat)

    negA3 = negA.reshape(_BSZ, 1, _K)
    negB3 = negB.reshape(_BSZ, 1, _K)
    d2 = dmat.reshape(_BSZ, _MIX, _BSZ)
    cidx3 = contrast_idx.astype(jnp.float32).reshape(_BSZ, 1, _K)
    ng_row = _tcb(negA3, negB3, d2, cidx3, idx_col)

    loss = _tcc(posx.reshape(1, _BSZ), ng_row)
    return loss.reshape(())


# speedup vs baseline: 2.6853x; 1.5319x over previous
import functools

import jax
import jax.numpy as jnp
from jax import lax
from jax.experimental import pallas as pl
from jax.experimental.pallas import tpu as pltpu
from jax.experimental.pallas import tpu_sc as plsc

_BSZ = 128
_MIX = 2
_FEAT = 128
_K = 2048
_TEMP = 0.07
_EPS = 0.1
_MOM = 0.5
_ITERS = 100

_NW = 32
_TOT = _BSZ * _K
_PER_W = _TOT // _NW
_CH = 128
_B_PER_W = _BSZ // _NW
_CH_PER_B = _K // _CH


def _mesh():
    return plsc.VectorSubcoreMesh(core_axis_name="c", subcore_axis_name="s")


def _sc_midx_body(mem_hbm, idx_hbm, midx_hbm, idxbuf, rows, sem):
    c = lax.axis_index("c")
    s = lax.axis_index("s")
    wid = s * 2 + c

    @pl.when(wid < 16)
    def _():
        base = pl.multiple_of(wid * 8, 8)
        pltpu.sync_copy(idx_hbm.at[pl.ds(base, 8)], idxbuf)
        pltpu.async_copy(mem_hbm.at[idxbuf], rows, sem).wait()
        pltpu.sync_copy(rows, midx_hbm.at[pl.ds(base, 8)])


def _sc_midx(memory, idx):
    f = pl.kernel(
        _sc_midx_body,
        mesh=_mesh(),
        out_type=jax.ShapeDtypeStruct((_BSZ, _FEAT), jnp.float32),
        scratch_types=[
            pltpu.VMEM((8,), jnp.int32),
            pltpu.VMEM((8, _FEAT), jnp.float32),
            pltpu.SemaphoreType.DMA,
        ],
    )
    return f(memory, idx)


_NBLK = 4096
_NSTEPS = (100000 + _NBLK - 1) // _NBLK


def _tcd_body(es_ref, mem_ref, d_ref, idxc_ref, out_ref):
    i = pl.program_id(0)
    logits = _dotT(es_ref[...], mem_ref[...])
    cols = ((i * _NBLK).astype(jnp.float32) +
            lax.broadcasted_iota(jnp.int32, (1, _NBLK), 1).astype(jnp.float32))
    onehot = (idxc_ref[...] == cols).astype(jnp.float32)
    corr = lax.dot_general(d_ref[...], onehot, (((1,), (0,)), ((), ())),
                           preferred_element_type=jnp.float32)
    out_ref[...] = logits + corr


def _tcd(es, memory, dmat, idx_col):
    return pl.pallas_call(
        _tcd_body,
        grid=(_NSTEPS,),
        in_specs=[
            pl.BlockSpec((_BSZ * _MIX, _FEAT), lambda i: (0, 0)),
            pl.BlockSpec((_NBLK, _FEAT), lambda i: (i, 0)),
            pl.BlockSpec((_BSZ * _MIX, _BSZ), lambda i: (0, 0)),
            pl.BlockSpec((_BSZ, 1), lambda i: (0, 0)),
        ],
        out_specs=pl.BlockSpec((_BSZ * _MIX, _NBLK), lambda i: (0, i)),
        out_shape=jax.ShapeDtypeStruct((_BSZ * _MIX, 100000), jnp.float32),
        compiler_params=pltpu.CompilerParams(
            dimension_semantics=("arbitrary",)),
    )(es, memory, dmat, idx_col)


def _lanes(val):
    return jnp.zeros((16,), jnp.int32) + val


def _sc_neggather_body(lg_hbm, cidx_hbm, negA_hbm, negB_hbm,
                       cbuf, abuf, bbuf, rA, rB, semA, semB):
    c = lax.axis_index("c")
    s = lax.axis_index("s")
    wid = s * 2 + c
    base_chunk = wid * (_PER_W // _CH)

    def start(j):
        p = lax.rem(j, 2)
        off = pl.multiple_of((base_chunk + j) * _CH, _CH)
        b = lax.div(base_chunk + j, _CH_PER_B)
        base_a = 2 * b * 100000
        csl = cbuf.at[pl.ds(p * _CH, _CH)]
        pltpu.sync_copy(cidx_hbm.at[pl.ds(off, _CH)], csl)
        for g in range(8):
            v = cbuf[pl.ds(p * _CH + g * 16, 16)]
            abuf[pl.ds(p * _CH + g * 16, 16)] = v + base_a
            bbuf[pl.ds(p * _CH + g * 16, 16)] = v + (base_a + 100000)
        pltpu.async_copy(lg_hbm.at[abuf.at[pl.ds(p * _CH, _CH)]],
                         rA.at[pl.ds(p * _CH, _CH)], semA.at[p])
        pltpu.async_copy(lg_hbm.at[bbuf.at[pl.ds(p * _CH, _CH)]],
                         rB.at[pl.ds(p * _CH, _CH)], semB.at[p])

    start(jnp.int32(0))

    def chunk_body(j, carry):
        @pl.when(j < _PER_W // _CH - 1)
        def _():
            start(j + 1)
        p = lax.rem(j, 2)
        off = pl.multiple_of((base_chunk + j) * _CH, _CH)
        dummy = lg_hbm.at[pl.ds(0, _CH)]
        pltpu.make_async_copy(dummy, rA.at[pl.ds(p * _CH, _CH)],
                              semA.at[p]).wait()
        pltpu.make_async_copy(dummy, rB.at[pl.ds(p * _CH, _CH)],
                              semB.at[p]).wait()
        pltpu.sync_copy(rA.at[pl.ds(p * _CH, _CH)],
                        negA_hbm.at[pl.ds(off, _CH)])
        pltpu.sync_copy(rB.at[pl.ds(p * _CH, _CH)],
                        negB_hbm.at[pl.ds(off, _CH)])
        return carry

    lax.fori_loop(0, _PER_W // _CH, chunk_body, 0)


def _sc_neggather(logits_flat, cidx_flat):
    f = pl.kernel(
        _sc_neggather_body,
        mesh=_mesh(),
        out_type=[
            jax.ShapeDtypeStruct((_TOT,), jnp.float32),
            jax.ShapeDtypeStruct((_TOT,), jnp.float32),
        ],
        scratch_types=[
            pltpu.VMEM((2 * _CH,), jnp.int32),
            pltpu.VMEM((2 * _CH,), jnp.int32),
            pltpu.VMEM((2 * _CH,), jnp.int32),
            pltpu.VMEM((2 * _CH,), jnp.float32),
            pltpu.VMEM((2 * _CH,), jnp.float32),
            pltpu.SemaphoreType.DMA((2,)),
            pltpu.SemaphoreType.DMA((2,)),
        ],
    )
    return f(logits_flat, cidx_flat)


def _dotT(a, b):
    return lax.dot_general(a, b, (((1,), (1,)), ((), ())),
                           preferred_element_type=jnp.float32)


def _tca_body(fs_ref, ft_ref, wsw_ref, wsb_ref, wtw_ref, wtb_ref,
              midx_ref, idxr_ref, idxc_ref,
              es_ref, d_ref, posx_ref):
    fs = fs_ref[...]
    ft = ft_ref[...]
    es = _dotT(fs, wsw_ref[...]) + wsb_ref[...]
    et = _dotT(ft, wtw_ref[...]) + wtb_ref[...]
    es = es * jax.lax.rsqrt(jnp.sum(es * es, axis=1, keepdims=True))
    et = et * jax.lax.rsqrt(jnp.sum(et * et, axis=1, keepdims=True))

    ii = lax.broadcasted_iota(jnp.int32, (_BSZ, _BSZ * _MIX), 0)
    jj = lax.broadcasted_iota(jnp.int32, (_BSZ, _BSZ * _MIX), 1)
    sel_e = (jj == 2 * ii).astype(jnp.float32)
    sel_o = (jj == 2 * ii + 1).astype(jnp.float32)
    es_e = lax.dot_general(sel_e, es, (((1,), (0,)), ((), ())),
                           preferred_element_type=jnp.float32)
    es_o = lax.dot_general(sel_o, es, (((1,), (0,)), ((), ())),
                           preferred_element_type=jnp.float32)
    et_e = lax.dot_general(sel_e, et, (((1,), (0,)), ((), ())),
                           preferred_element_type=jnp.float32)
    et_o = lax.dot_general(sel_o, et, (((1,), (0,)), ((), ())),
                           preferred_element_type=jnp.float32)

    midx = midx_ref[...]
    pos = midx * _MOM + et_e * (1.0 - _MOM)
    pos = pos * jax.lax.rsqrt(jnp.sum(pos * pos, axis=1, keepdims=True))

    idx_r = idxr_ref[...]
    idx_c = idxc_ref[...]
    eqm = (idx_c == idx_r).astype(jnp.float32)
    pp = lax.broadcasted_iota(jnp.int32, (_BSZ, _BSZ), 0)
    qq = lax.broadcasted_iota(jnp.int32, (_BSZ, _BSZ), 1)
    later_dup = eqm * (qq > pp).astype(jnp.float32)
    active = 1.0 - jnp.max(later_dup, axis=1, keepdims=True)

    delta = (pos - midx) * active
    d_ref[...] = _dotT(es, delta)

    ones_row = jnp.ones((1, _FEAT), jnp.float32)

    def _rowdot(x, y):
        return lax.dot_general(ones_row, x * y, (((1,), (1,)), ((), ())),
                               preferred_element_type=jnp.float32)

    g00 = _rowdot(es_e, et_e)
    g01 = _rowdot(es_e, et_o)
    g10 = _rowdot(es_o, et_e)
    g11 = _rowdot(es_o, et_o)
    c00 = 2.0 - 2.0 * g00
    c01 = 2.0 - 2.0 * g01
    c10 = 2.0 - 2.0 * g10
    c11 = 2.0 - 2.0 * g11
    lmu = jnp.log(0.5 + 1e-8)

    def m_all(u0, u1, v0, v1):
        m00 = (-c00 + u0 + v0) / _EPS
        m01 = (-c01 + u0 + v1) / _EPS
        m10 = (-c10 + u1 + v0) / _EPS
        m11 = (-c11 + u1 + v1) / _EPS
        return m00, m01, m10, m11

    def sink_step(_, carry):
        u0, u1, v0, v1 = carry
        m00, m01, m10, m11 = m_all(u0, u1, v0, v1)
        u0 = _EPS * (lmu - jnp.logaddexp(m00, m01)) + u0
        u1 = _EPS * (lmu - jnp.logaddexp(m10, m11)) + u1
        m00, m01, m10, m11 = m_all(u0, u1, v0, v1)
        v0 = _EPS * (lmu - jnp.logaddexp(m00, m10)) + v0
        v1 = _EPS * (lmu - jnp.logaddexp(m01, m11)) + v1
        return u0, u1, v0, v1

    z = jnp.zeros((1, _BSZ), jnp.float32)
    u0, u1, v0, v1 = lax.fori_loop(0, _ITERS, sink_step, (z, z, z, z))
    m00, m01, m10, m11 = m_all(u0, u1, v0, v1)
    posx = (jnp.exp(m00) * g00 + jnp.exp(m01) * g01 +
            jnp.exp(m10) * g10 + jnp.exp(m11) * g11)

    es_ref[...] = es
    posx_ref[...] = posx


def _tca(feat_s, feat_t, wsw, wsb, wtw, wtb, midx, idx_row, idx_col):
    return pl.pallas_call(
        _tca_body,
        out_shape=[
            jax.ShapeDtypeStruct((_BSZ * _MIX, _FEAT), jnp.float32),
            jax.ShapeDtypeStruct((_BSZ * _MIX, _BSZ), jnp.float32),
            jax.ShapeDtypeStruct((1, _BSZ), jnp.float32),
        ],
    )(feat_s, feat_t, wsw, wsb, wtw, wtb, midx, idx_row, idx_col)


def _tce_body(negA_ref, negB_ref, posx_ref, out_ref):
    ng = (jnp.exp(negA_ref[...] / _TEMP) +
          jnp.exp(negB_ref[...] / _TEMP))
    ones_k = jnp.ones((1, _K), jnp.float32)
    ngs = lax.dot_general(ones_k, ng, (((1,), (1,)), ((), ())),
                          preferred_element_type=jnp.float32)
    p = jnp.exp(posx_ref[...] / _TEMP)
    logits = jnp.log(p / (p + ngs))
    out_ref[0, 0] = -jnp.sum(logits) / _BSZ


def _tce(negA, negB, posx_row):
    return pl.pallas_call(
        _tce_body,
        out_specs=pl.BlockSpec(memory_space=pltpu.SMEM),
        out_shape=jax.ShapeDtypeStruct((1, 1), jnp.float32),
    )(negA, negB, posx_row)


def kernel(feat_s, feat_t, memory, Ws_w, Ws_b, Wt_w, Wt_b, labels, idx,
           contrast_idx):
    feat_s = feat_s.reshape(_BSZ * _MIX, -1)
    feat_t = feat_t.reshape(_BSZ * _MIX, -1)
    cidx_flat = contrast_idx.reshape(_TOT).astype(jnp.int32)
    idx_i = idx.astype(jnp.int32)

    midx = _sc_midx(memory, idx_i)

    idx_f = idx.astype(jnp.float32)
    idx_row = idx_f.reshape(1, _BSZ)
    idx_col = idx_f.reshape(_BSZ, 1)
    es, dmat, posx = _tca(feat_s, feat_t, Ws_w, Ws_b.reshape(1, _FEAT),
                          Wt_w, Wt_b.reshape(1, _FEAT), midx, idx_row, idx_col)

    logits = _tcd(es, memory, dmat, idx_col)
    negA, negB = _sc_neggather(logits.reshape(_BSZ * _MIX * 100000), cidx_flat)

    loss = _tce(negA.reshape(_BSZ, _K), negB.reshape(_BSZ, _K), posx)
    return loss.reshape(())


# speedup vs baseline: 3.5398x; 1.3182x over previous
import functools

import jax
import jax.numpy as jnp
from jax import lax
from jax.experimental import pallas as pl
from jax.experimental.pallas import tpu as pltpu
from jax.experimental.pallas import tpu_sc as plsc

_BSZ = 128
_MIX = 2
_FEAT = 128
_K = 2048
_TEMP = 0.07
_EPS = 0.1
_MOM = 0.5
_ITERS = 100

_NW = 32
_TOT = _BSZ * _K
_PER_W = _TOT // _NW
_CH = 128
_B_PER_W = _BSZ // _NW
_CH_PER_B = _K // _CH


def _mesh():
    return plsc.VectorSubcoreMesh(core_axis_name="c", subcore_axis_name="s")


def _sc_midx_body(mem_hbm, idx_hbm, midx_hbm, idxbuf, rows, sem):
    c = lax.axis_index("c")
    s = lax.axis_index("s")
    wid = s * 2 + c

    @pl.when(wid < 16)
    def _():
        base = pl.multiple_of(wid * 8, 8)
        pltpu.sync_copy(idx_hbm.at[pl.ds(base, 8)], idxbuf)
        pltpu.async_copy(mem_hbm.at[idxbuf], rows, sem).wait()
        pltpu.sync_copy(rows, midx_hbm.at[pl.ds(base, 8)])


def _sc_midx(memory, idx):
    f = pl.kernel(
        _sc_midx_body,
        mesh=_mesh(),
        out_type=jax.ShapeDtypeStruct((_BSZ, _FEAT), jnp.float32),
        scratch_types=[
            pltpu.VMEM((8,), jnp.int32),
            pltpu.VMEM((8, _FEAT), jnp.float32),
            pltpu.SemaphoreType.DMA,
        ],
    )
    return f(memory, idx)


_NBLK = 4096
_NSTEPS = (100000 + _NBLK - 1) // _NBLK


def _tcd_body(es_ref, mem_ref, d_ref, idxr_ref, out_ref):
    i = pl.program_id(0)
    logits = _dotT(mem_ref[...], es_ref[...])
    rowids = ((i * _NBLK).astype(jnp.float32) +
              lax.broadcasted_iota(jnp.int32, (_NBLK, 1), 0)
              .astype(jnp.float32))
    onehot = (rowids == idxr_ref[...]).astype(jnp.float32)
    corr = lax.dot_general(onehot, d_ref[...], (((1,), (1,)), ((), ())),
                           preferred_element_type=jnp.float32)
    out_ref[...] = logits + corr


def _tcd(es, memory, dmat, idx_row):
    return pl.pallas_call(
        _tcd_body,
        grid=(_NSTEPS,),
        in_specs=[
            pl.BlockSpec((_BSZ * _MIX, _FEAT), lambda i: (0, 0)),
            pl.BlockSpec((_NBLK, _FEAT), lambda i: (i, 0)),
            pl.BlockSpec((_BSZ * _MIX, _BSZ), lambda i: (0, 0)),
            pl.BlockSpec((1, _BSZ), lambda i: (0, 0)),
        ],
        out_specs=pl.BlockSpec((_NBLK, _BSZ * _MIX), lambda i: (i, 0)),
        out_shape=jax.ShapeDtypeStruct((100000, _BSZ * _MIX), jnp.float32),
        compiler_params=pltpu.CompilerParams(
            dimension_semantics=("arbitrary",)),
    )(es, memory, dmat, idx_row)


def _lanes(val):
    return jnp.zeros((16,), jnp.int32) + val


def _sc_neggather_body(lg_hbm, cidx_hbm, negA_hbm, negB_hbm,
                       cbuf, abuf, bbuf, rA, rB, semA, semB):
    c = lax.axis_index("c")
    s = lax.axis_index("s")
    wid = s * 2 + c
    base_chunk = wid * (_PER_W // _CH)

    def start(j):
        p = lax.rem(j, 2)
        off = pl.multiple_of((base_chunk + j) * _CH, _CH)
        b = lax.div(base_chunk + j, _CH_PER_B)
        csl = cbuf.at[pl.ds(p * _CH, _CH)]
        pltpu.sync_copy(cidx_hbm.at[pl.ds(off, _CH)], csl)
        for g in range(8):
            v = cbuf[pl.ds(p * _CH + g * 16, 16)]
            abuf[pl.ds(p * _CH + g * 16, 16)] = v * (_BSZ * _MIX) + 2 * b
            bbuf[pl.ds(p * _CH + g * 16, 16)] = v * (_BSZ * _MIX) + 2 * b + 1
        pltpu.async_copy(lg_hbm.at[abuf.at[pl.ds(p * _CH, _CH)]],
                         rA.at[pl.ds(p * _CH, _CH)], semA.at[p])
        pltpu.async_copy(lg_hbm.at[bbuf.at[pl.ds(p * _CH, _CH)]],
                         rB.at[pl.ds(p * _CH, _CH)], semB.at[p])

    start(jnp.int32(0))

    def chunk_body(j, carry):
        @pl.when(j < _PER_W // _CH - 1)
        def _():
            start(j + 1)
        p = lax.rem(j, 2)
        off = pl.multiple_of((base_chunk + j) * _CH, _CH)
        dummy = lg_hbm.at[pl.ds(0, _CH)]
        pltpu.make_async_copy(dummy, rA.at[pl.ds(p * _CH, _CH)],
                              semA.at[p]).wait()
        pltpu.make_async_copy(dummy, rB.at[pl.ds(p * _CH, _CH)],
                              semB.at[p]).wait()
        pltpu.sync_copy(rA.at[pl.ds(p * _CH, _CH)],
                        negA_hbm.at[pl.ds(off, _CH)])
        pltpu.sync_copy(rB.at[pl.ds(p * _CH, _CH)],
                        negB_hbm.at[pl.ds(off, _CH)])
        return carry

    lax.fori_loop(0, _PER_W // _CH, chunk_body, 0)


def _sc_neggather(logits_flat, cidx_flat):
    f = pl.kernel(
        _sc_neggather_body,
        mesh=_mesh(),
        out_type=[
            jax.ShapeDtypeStruct((_TOT,), jnp.float32),
            jax.ShapeDtypeStruct((_TOT,), jnp.float32),
        ],
        scratch_types=[
            pltpu.VMEM((2 * _CH,), jnp.int32),
            pltpu.VMEM((2 * _CH,), jnp.int32),
            pltpu.VMEM((2 * _CH,), jnp.int32),
            pltpu.VMEM((2 * _CH,), jnp.float32),
            pltpu.VMEM((2 * _CH,), jnp.float32),
            pltpu.SemaphoreType.DMA((2,)),
            pltpu.SemaphoreType.DMA((2,)),
        ],
    )
    return f(logits_flat, cidx_flat)


def _dotT(a, b):
    return lax.dot_general(a, b, (((1,), (1,)), ((), ())),
                           preferred_element_type=jnp.float32)


def _tca_body(fs_ref, ft_ref, wsw_ref, wsb_ref, wtw_ref, wtb_ref,
              midx_ref, idxr_ref, idxc_ref,
              es_ref, d_ref, posx_ref):
    fs = fs_ref[...]
    ft = ft_ref[...]
    es = _dotT(fs, wsw_ref[...]) + wsb_ref[...]
    et = _dotT(ft, wtw_ref[...]) + wtb_ref[...]
    es = es * jax.lax.rsqrt(jnp.sum(es * es, axis=1, keepdims=True))
    et = et * jax.lax.rsqrt(jnp.sum(et * et, axis=1, keepdims=True))

    ii = lax.broadcasted_iota(jnp.int32, (_BSZ, _BSZ * _MIX), 0)
    jj = lax.broadcasted_iota(jnp.int32, (_BSZ, _BSZ * _MIX), 1)
    sel_e = (jj == 2 * ii).astype(jnp.float32)
    sel_o = (jj == 2 * ii + 1).astype(jnp.float32)
    es_e = lax.dot_general(sel_e, es, (((1,), (0,)), ((), ())),
                           preferred_element_type=jnp.float32)
    es_o = lax.dot_general(sel_o, es, (((1,), (0,)), ((), ())),
                           preferred_element_type=jnp.float32)
    et_e = lax.dot_general(sel_e, et, (((1,), (0,)), ((), ())),
                           preferred_element_type=jnp.float32)
    et_o = lax.dot_general(sel_o, et, (((1,), (0,)), ((), ())),
                           preferred_element_type=jnp.float32)

    midx = midx_ref[...]
    pos = midx * _MOM + et_e * (1.0 - _MOM)
    pos = pos * jax.lax.rsqrt(jnp.sum(pos * pos, axis=1, keepdims=True))

    idx_r = idxr_ref[...]
    idx_c = idxc_ref[...]
    eqm = (idx_c == idx_r).astype(jnp.float32)
    pp = lax.broadcasted_iota(jnp.int32, (_BSZ, _BSZ), 0)
    qq = lax.broadcasted_iota(jnp.int32, (_BSZ, _BSZ), 1)
    later_dup = eqm * (qq > pp).astype(jnp.float32)
    active = 1.0 - jnp.max(later_dup, axis=1, keepdims=True)

    delta = (pos - midx) * active
    d_ref[...] = _dotT(es, delta)

    ones_row = jnp.ones((1, _FEAT), jnp.float32)

    def _rowdot(x, y):
        return lax.dot_general(ones_row, x * y, (((1,), (1,)), ((), ())),
                               preferred_element_type=jnp.float32)

    g00 = _rowdot(es_e, et_e)
    g01 = _rowdot(es_e, et_o)
    g10 = _rowdot(es_o, et_e)
    g11 = _rowdot(es_o, et_o)
    c00 = 2.0 - 2.0 * g00
    c01 = 2.0 - 2.0 * g01
    c10 = 2.0 - 2.0 * g10
    c11 = 2.0 - 2.0 * g11
    lmu = jnp.log(0.5 + 1e-8)

    def m_all(u0, u1, v0, v1):
        m00 = (-c00 + u0 + v0) / _EPS
        m01 = (-c01 + u0 + v1) / _EPS
        m10 = (-c10 + u1 + v0) / _EPS
        m11 = (-c11 + u1 + v1) / _EPS
        return m00, m01, m10, m11

    def sink_step(_, carry):
        u0, u1, v0, v1 = carry
        m00, m01, m10, m11 = m_all(u0, u1, v0, v1)
        u0 = _EPS * (lmu - jnp.logaddexp(m00, m01)) + u0
        u1 = _EPS * (lmu - jnp.logaddexp(m10, m11)) + u1
        m00, m01, m10, m11 = m_all(u0, u1, v0, v1)
        v0 = _EPS * (lmu - jnp.logaddexp(m00, m10)) + v0
        v1 = _EPS * (lmu - jnp.logaddexp(m01, m11)) + v1
        return u0, u1, v0, v1

    z = jnp.zeros((1, _BSZ), jnp.float32)
    u0, u1, v0, v1 = lax.fori_loop(0, _ITERS, sink_step, (z, z, z, z))
    m00, m01, m10, m11 = m_all(u0, u1, v0, v1)
    posx = (jnp.exp(m00) * g00 + jnp.exp(m01) * g01 +
            jnp.exp(m10) * g10 + jnp.exp(m11) * g11)

    es_ref[...] = es
    posx_ref[...] = posx


def _tca(feat_s, feat_t, wsw, wsb, wtw, wtb, midx, idx_row, idx_col):
    return pl.pallas_call(
        _tca_body,
        out_shape=[
            jax.ShapeDtypeStruct((_BSZ * _MIX, _FEAT), jnp.float32),
            jax.ShapeDtypeStruct((_BSZ * _MIX, _BSZ), jnp.float32),
            jax.ShapeDtypeStruct((1, _BSZ), jnp.float32),
        ],
    )(feat_s, feat_t, wsw, wsb, wtw, wtb, midx, idx_row, idx_col)


def _tce_body(negA_ref, negB_ref, posx_ref, out_ref):
    ng = (jnp.exp(negA_ref[...] / _TEMP) +
          jnp.exp(negB_ref[...] / _TEMP))
    ones_k = jnp.ones((1, _K), jnp.float32)
    ngs = lax.dot_general(ones_k, ng, (((1,), (1,)), ((), ())),
                          preferred_element_type=jnp.float32)
    p = jnp.exp(posx_ref[...] / _TEMP)
    logits = jnp.log(p / (p + ngs))
    out_ref[0, 0] = -jnp.sum(logits) / _BSZ


def _tce(negA, negB, posx_row):
    return pl.pallas_call(
        _tce_body,
        out_specs=pl.BlockSpec(memory_space=pltpu.SMEM),
        out_shape=jax.ShapeDtypeStruct((1, 1), jnp.float32),
    )(negA, negB, posx_row)


def kernel(feat_s, feat_t, memory, Ws_w, Ws_b, Wt_w, Wt_b, labels, idx,
           contrast_idx):
    feat_s = feat_s.reshape(_BSZ * _MIX, -1)
    feat_t = feat_t.reshape(_BSZ * _MIX, -1)
    cidx_flat = contrast_idx.reshape(_TOT).astype(jnp.int32)
    idx_i = idx.astype(jnp.int32)

    midx = _sc_midx(memory, idx_i)

    idx_f = idx.astype(jnp.float32)
    idx_row = idx_f.reshape(1, _BSZ)
    idx_col = idx_f.reshape(_BSZ, 1)
    es, dmat, posx = _tca(feat_s, feat_t, Ws_w, Ws_b.reshape(1, _FEAT),
                          Wt_w, Wt_b.reshape(1, _FEAT), midx, idx_row, idx_col)

    logits = _tcd(es, memory, dmat, idx_row)
    negA, negB = _sc_neggather(logits.reshape(100000 * _BSZ * _MIX), cidx_flat)

    loss = _tce(negA.reshape(_BSZ, _K), negB.reshape(_BSZ, _K), posx)
    return loss.reshape(())
